# baseline (speedup 1.0000x reference)
import functools

import jax
import jax.numpy as jnp
from jax import lax
from jax.experimental import pallas as pl
from jax.experimental.pallas import tpu as pltpu
from jax.experimental.pallas import tpu_sc as plsc


_ROWS_PER_W = 512
_W_PER_B = 8
_GRP = 16
_BIG = 3.0e38


def _gather16(v, idx):
    dnums = lax.GatherDimensionNumbers(
        offset_dims=(), collapsed_slice_dims=(0,), start_index_map=(0,))
    return lax.gather(v, idx[:, None], dnums, (1,),
                      mode=lax.GatherScatterMode.PROMISE_IN_BOUNDS)


def _sc_body(xt_hbm, yt_hbm, out1_hbm, out2_hbm, xbuf, ybuf, d1buf, d2buf):
    c = lax.axis_index("c")
    s = lax.axis_index("s")
    wid = s * 2 + c
    b = wid // _W_PER_B
    r = wid % _W_PER_B
    row0 = r * _ROWS_PER_W

    pltpu.sync_copy(xt_hbm.at[b, :, pl.ds(row0, _ROWS_PER_W)], xbuf)
    pltpu.sync_copy(yt_hbm.at[b], ybuf)

    def init_chunk(j, carry):
        d2buf[pl.ds(j * 16, 16)] = jnp.full((16,), _BIG, jnp.float32)
        return carry

    lax.fori_loop(0, 256, init_chunk, 0)

    lane = lax.iota(jnp.int32, 16)

    def row_group(g, carry):
        vx0 = xbuf[0, pl.ds(g * _GRP, _GRP)]
        vx1 = xbuf[1, pl.ds(g * _GRP, _GRP)]
        vx2 = xbuf[2, pl.ds(g * _GRP, _GRP)]
        xs = [(vx0[q], vx1[q], vx2[q]) for q in range(_GRP)]

        def chunk(j, rms):
            y0 = ybuf[0, pl.ds(j * 16, 16)]
            y1 = ybuf[1, pl.ds(j * 16, 16)]
            y2 = ybuf[2, pl.ds(j * 16, 16)]
            d2v = d2buf[pl.ds(j * 16, 16)]
            new_rms = []
            for q in range(_GRP):
                t0 = xs[q][0] - y0
                t1 = xs[q][1] - y1
                t2 = xs[q][2] - y2
                d = t0 * t0 + t1 * t1 + t2 * t2
                d2v = jnp.minimum(d2v, d)
                new_rms.append(jnp.minimum(rms[q], d))
            d2buf[pl.ds(j * 16, 16)] = d2v
            return tuple(new_rms)

        init = tuple(jnp.full((16,), _BIG, jnp.float32) for _ in range(_GRP))
        rms = lax.fori_loop(0, 256, chunk, init)

        d1v = jnp.full((16,), _BIG, jnp.float32)
        for q in range(_GRP):
            v = rms[q]
            for sh in (8, 4, 2, 1):
                v = jnp.minimum(v, _gather16(v, lane ^ sh))
            d1v = jnp.where(lane == q, v, d1v)
        d1buf[pl.ds(g * _GRP, _GRP)] = d1v
        return carry

    lax.fori_loop(0, _ROWS_PER_W // _GRP, row_group, 0)

    pltpu.sync_copy(d1buf, out1_hbm.at[b, pl.ds(row0, _ROWS_PER_W)])
    pltpu.sync_copy(d2buf, out2_hbm.at[b, r])


def kernel(input1, input2):
    B, N, _ = input1.shape
    M = input2.shape[1]
    xt = jnp.transpose(input1, (0, 2, 1))
    yt = jnp.transpose(input2, (0, 2, 1))

    mesh = plsc.VectorSubcoreMesh(core_axis_name="c", subcore_axis_name="s")
    run = functools.partial(
        pl.kernel,
        mesh=mesh,
        out_type=[
            jax.ShapeDtypeStruct((B, N), jnp.float32),
            jax.ShapeDtypeStruct((B, _W_PER_B, M), jnp.float32),
        ],
        scratch_types=[
            pltpu.VMEM((3, _ROWS_PER_W), jnp.float32),
            pltpu.VMEM((3, M), jnp.float32),
            pltpu.VMEM((_ROWS_PER_W,), jnp.float32),
            pltpu.VMEM((M,), jnp.float32),
        ],
    )(_sc_body)
    out1, out2p = run(xt, yt)
    return out1, jnp.min(out2p, axis=1)


# speedup vs baseline: 13.6984x; 13.6984x over previous
import jax
import jax.numpy as jnp
from jax.experimental import pallas as pl


_N_BLK = 2048


def _nnd_body(x_ref, yt_ref, d1_ref, d2_ref):
    nb = pl.program_id(1)
    x = x_ref[0]
    yt = yt_ref[0]

    n_blk = x.shape[0]
    m = yt.shape[1]
    bf16, f32 = jnp.bfloat16, jnp.float32

    xm = ((-2.0) * x).astype(bf16)
    yb = yt.astype(bf16)
    x2 = jnp.sum(x * x, axis=1, keepdims=True)
    y2 = jnp.sum(yt * yt, axis=0, keepdims=True)
    x2h = x2.astype(bf16)
    x2l = (x2 - x2h.astype(f32)).astype(bf16)
    y2h = y2.astype(bf16)
    y2l = (y2 - y2h.astype(f32)).astype(bf16)

    lhs = jnp.concatenate(
        [xm, x2h, x2l,
         jnp.ones((n_blk, 1), bf16), jnp.ones((n_blk, 1), bf16)], axis=1)
    rhs = jnp.concatenate(
        [yb, jnp.ones((1, m), bf16), jnp.ones((1, m), bf16),
         y2h, y2l], axis=0)

    d = jax.lax.dot_general(lhs, rhs, (((1,), (0,)), ((), ())),
                            preferred_element_type=f32)

    d1_ref[0] = jnp.maximum(jnp.min(d, axis=1, keepdims=True), 0.0)

    cur = jnp.maximum(jnp.min(d, axis=0, keepdims=True), 0.0)

    @pl.when(nb == 0)
    def _init():
        d2_ref[0] = cur

    @pl.when(nb != 0)
    def _accum():
        d2_ref[0] = jnp.minimum(d2_ref[0], cur)


def kernel(input1, input2):
    B, N, _ = input1.shape
    M = input2.shape[1]
    yt = jnp.transpose(input2, (0, 2, 1))

    nb = N // _N_BLK
    out1, out2 = pl.pallas_call(
        _nnd_body,
        grid=(B, nb),
        in_specs=[
            pl.BlockSpec((1, _N_BLK, 3), lambda b, i: (b, i, 0)),
            pl.BlockSpec((1, 3, M), lambda b, i: (b, 0, 0)),
        ],
        out_specs=[
            pl.BlockSpec((1, _N_BLK, 1), lambda b, i: (b, i, 0)),
            pl.BlockSpec((1, 1, M), lambda b, i: (b, 0, 0)),
        ],
        out_shape=[
            jax.ShapeDtypeStruct((B, N, 1), jnp.float32),
            jax.ShapeDtypeStruct((B, 1, M), jnp.float32),
        ],
    )(input1, yt)
    return out1.reshape(B, N), out2.reshape(B, M)
